# baseline (speedup 1.0000x reference)
import functools

import jax
import jax.numpy as jnp
from jax import lax
from jax.experimental import pallas as pl
from jax.experimental.pallas import tpu as pltpu
from jax.experimental.pallas import tpu_sc as plsc

N_NODES = 10000
N_EDGES = 320000
H = 128
NEG = 0.1

NC = 2
NS = 16
NW = NC * NS
COLS = H // NW
XN = 10016

CH = 128
W_CHUNKS = 80
CHUNKS_PAD = 2560
E_PAD = CHUNKS_PAD * CH
IDX_SLAB = 64
N_SLABS = CHUNKS_PAD // IDX_SLAB
GROUPS = CH // 16


def _lrelu(v):
    return jnp.where(v >= 0, v, NEG * v)


def _colsum_body(xT_hbm, src_hbm, dst_hbm, outT_hbm,
                 xslab, acc, s0, s1, d0, d1,
                 sem_s0, sem_s1, sem_d0, sem_d1):
    c = lax.axis_index("c")
    s = lax.axis_index("s")
    q = s * NC + c

    pltpu.sync_copy(xT_hbm.at[q], xslab)

    zero16 = jnp.zeros((16,), jnp.float32)

    def _z(i, _):
        for cc in range(COLS):
            acc[cc, pl.ds(i * 16, 16)] = zero16
        return 0

    lax.fori_loop(0, XN // 16, _z, 0)

    def _fire(slab_idx, sbuf, dbuf, sem_s, sem_d):
        off = pl.multiple_of(slab_idx * IDX_SLAB, IDX_SLAB)
        pltpu.async_copy(src_hbm.at[pl.ds(off, IDX_SLAB)], sbuf, sem_s)
        pltpu.async_copy(dst_hbm.at[pl.ds(off, IDX_SLAB)], dbuf, sem_d)

    def _wait(sbuf, dbuf, sem_s, sem_d):
        pltpu.make_async_copy(src_hbm.at[pl.ds(0, IDX_SLAB)], sbuf,
                              sem_s).wait()
        pltpu.make_async_copy(dst_hbm.at[pl.ds(0, IDX_SLAB)], dbuf,
                              sem_d).wait()

    col_ids = [jnp.full((16,), cc, jnp.int32) for cc in range(COLS)]

    def _process(sbuf, dbuf):
        def chunk(k, _):
            for g in range(GROUPS):
                srcv = sbuf[k, pl.ds(g * 16, 16)]
                dstv = dbuf[k, pl.ds(g * 16, 16)]
                for cc in range(COLS):
                    vals = plsc.load_gather(xslab, [col_ids[cc], srcv])
                    plsc.addupdate_scatter(acc, [col_ids[cc], dstv], vals)
            return 0

        lax.fori_loop(0, IDX_SLAB, chunk, 0)

    _fire(0, s0, d0, sem_s0, sem_d0)
    _fire(1, s1, d1, sem_s1, sem_d1)

    def pair(t, _):
        j = t * 2
        _wait(s0, d0, sem_s0, sem_d0)
        _process(s0, d0)

        @pl.when(t < N_SLABS // 2 - 1)
        def _():
            _fire(j + 2, s0, d0, sem_s0, sem_d0)

        _wait(s1, d1, sem_s1, sem_d1)
        _process(s1, d1)

        @pl.when(t < N_SLABS // 2 - 1)
        def _():
            _fire(j + 3, s1, d1, sem_s1, sem_d1)

        return 0

    lax.fori_loop(0, N_SLABS // 2, pair, 0)

    pltpu.sync_copy(acc, outT_hbm.at[q])


_colsum = pl.kernel(
    _colsum_body,
    out_type=jax.ShapeDtypeStruct((NW, COLS, XN), jnp.float32),
    mesh=plsc.VectorSubcoreMesh(core_axis_name="c", subcore_axis_name="s"),
    compiler_params=pltpu.CompilerParams(needs_layout_passes=False),
    scratch_types=[
        pltpu.VMEM((COLS, XN), jnp.float32),
        pltpu.VMEM((COLS, XN), jnp.float32),
        pltpu.VMEM((IDX_SLAB, CH), jnp.int32),
        pltpu.VMEM((IDX_SLAB, CH), jnp.int32),
        pltpu.VMEM((IDX_SLAB, CH), jnp.int32),
        pltpu.VMEM((IDX_SLAB, CH), jnp.int32),
        pltpu.SemaphoreType.DMA,
        pltpu.SemaphoreType.DMA,
        pltpu.SemaphoreType.DMA,
        pltpu.SemaphoreType.DMA,
    ],
)


def _embed_body(xg, xl, xo, xe,
                wg1, bg1, wg2, bg2, wl1, bl1, wl2, bl2,
                wo1, bo1, wo2, bo2, we1, be1, we2, be2, out):
    def mlp2(xt, w1, b1, w2, b2):
        h = _lrelu(jnp.dot(w1[...], xt[...],
                           preferred_element_type=jnp.float32) + b1[...])
        return _lrelu(jnp.dot(w2[...], h,
                              preferred_element_type=jnp.float32) + b2[...])

    out[:, 0:1000] = mlp2(xg, wg1, bg1, wg2, bg2)
    out[:, 1000:2000] = mlp2(xl, wl1, bl1, wl2, bl2)
    out[:, 2000:6000] = mlp2(xo, wo1, bo1, wo2, bo2)
    out[:, 6000:10000] = mlp2(xe, we1, be1, we2, be2)
    out[:, 10000:XN] = jnp.zeros((H, XN - N_NODES), jnp.float32)


_embed = pl.pallas_call(
    _embed_body,
    out_shape=jax.ShapeDtypeStruct((H, XN), jnp.float32),
)


def _layer_body(aggT, xT, wl, bl, wr, out):
    y = (jnp.dot(wl[...], aggT[...], preferred_element_type=jnp.float32)
         + bl[...]
         + jnp.dot(wr[...], xT[...], preferred_element_type=jnp.float32))
    out[...] = _lrelu(y)


_layer = pl.pallas_call(
    _layer_body,
    out_shape=jax.ShapeDtypeStruct((H, XN), jnp.float32),
)


def _final_body(aggT, xT, wl5t, bl, wr5t, out):
    dn = (((0,), (0,)), ((), ()))
    y = (lax.dot_general(aggT[:, 0:N_NODES], wl5t[...], dn,
                         preferred_element_type=jnp.float32)
         + bl[...]
         + lax.dot_general(xT[:, 0:N_NODES], wr5t[...], dn,
                           preferred_element_type=jnp.float32))
    out[...] = jax.nn.sigmoid(y)


_final = pl.pallas_call(
    _final_body,
    out_shape=jax.ShapeDtypeStruct((N_NODES, 1), jnp.float32),
)


def kernel(x_gen, x_load, x_or, x_ex, edge_index, object_ptv,
           W_gen1, b_gen1, W_gen2, b_gen2,
           W_load1, b_load1, W_load2, b_load2,
           W_or1, b_or1, W_or2, b_or2,
           W_ex1, b_ex1, W_ex2, b_ex2,
           Wl_0, bl_0, Wr_0, Wl_1, bl_1, Wr_1, Wl_2, bl_2, Wr_2,
           Wl_3, bl_3, Wr_3, Wl_4, bl_4, Wr_4, Wl_5, bl_5, Wr_5):
    npad = E_PAD - N_EDGES
    src2d = jnp.concatenate(
        [edge_index[0], jnp.zeros((npad,), jnp.int32)]).reshape(CHUNKS_PAD, CH)
    dst2d = jnp.concatenate(
        [edge_index[1], jnp.full((npad,), N_NODES, jnp.int32)]
    ).reshape(CHUNKS_PAD, CH)

    def t(w):
        return jnp.transpose(w)

    def b2(b):
        return b.reshape(-1, 1)

    xT = _embed(t(x_gen), t(x_load), t(x_or), t(x_ex),
                W_gen1, b2(b_gen1), W_gen2, b2(b_gen2),
                W_load1, b2(b_load1), W_load2, b2(b_load2),
                W_or1, b2(b_or1), W_or2, b2(b_or2),
                W_ex1, b2(b_ex1), W_ex2, b2(b_ex2))

    layers = [(Wl_0, bl_0, Wr_0), (Wl_1, bl_1, Wr_1), (Wl_2, bl_2, Wr_2),
              (Wl_3, bl_3, Wr_3), (Wl_4, bl_4, Wr_4)]
    for wl, bl, wr in layers:
        aggT3 = _colsum(xT.reshape(NW, COLS, XN), src2d, dst2d)
        xT = _layer(aggT3.reshape(H, XN), xT, wl, b2(bl), wr)

    aggT3 = _colsum(xT.reshape(NW, COLS, XN), src2d, dst2d)
    return _final(aggT3.reshape(H, XN), xT, t(Wl_5), b2(bl_5), t(Wr_5))


# speedup vs baseline: 6.6377x; 6.6377x over previous
import functools

import jax
import jax.numpy as jnp
from jax import lax
from jax.experimental import pallas as pl
from jax.experimental.pallas import tpu as pltpu
from jax.experimental.pallas import tpu_sc as plsc

N_NODES = 10000
N_EDGES = 320000
H = 128
NEG = 0.1

NC = 2
NS = 16
NW = NC * NS
COLS = H // NW
XN = 10016

CH = 128
W_CHUNKS = 80
CHUNKS_PAD = 2560
E_PAD = CHUNKS_PAD * CH
IDX_SLAB = 64
N_SLABS = CHUNKS_PAD // IDX_SLAB
GROUPS = CH // 16


def _lrelu(v):
    return jnp.where(v >= 0, v, NEG * v)


def _colsum_body(xT_hbm, src_hbm, dst_hbm, outT_hbm,
                 xslab, acc, s0, s1, d0, d1,
                 sem_s0, sem_s1, sem_d0, sem_d1):
    c = lax.axis_index("c")
    s = lax.axis_index("s")
    q = s * NC + c

    pltpu.sync_copy(xT_hbm.at[q], xslab)

    zero16 = jnp.zeros((16,), jnp.float32)

    def _z(i, _):
        for cc in range(COLS):
            acc[cc, pl.ds(i * 16, 16)] = zero16
        return 0

    lax.fori_loop(0, XN // 16, _z, 0)

    def _fire(slab_idx, sbuf, dbuf, sem_s, sem_d):
        off = pl.multiple_of(slab_idx * IDX_SLAB, IDX_SLAB)
        pltpu.async_copy(src_hbm.at[pl.ds(off, IDX_SLAB)], sbuf, sem_s)
        pltpu.async_copy(dst_hbm.at[pl.ds(off, IDX_SLAB)], dbuf, sem_d)

    def _wait(sbuf, dbuf, sem_s, sem_d):
        pltpu.make_async_copy(src_hbm.at[pl.ds(0, IDX_SLAB)], sbuf,
                              sem_s).wait()
        pltpu.make_async_copy(dst_hbm.at[pl.ds(0, IDX_SLAB)], dbuf,
                              sem_d).wait()

    col_ids = [jnp.full((16,), cc, jnp.int32) for cc in range(COLS)]

    def _process(sbuf, dbuf):
        @functools.partial(plsc.parallel_loop, 0, IDX_SLAB, unroll=2)
        def chunk(k):
            for g in range(GROUPS):
                srcv = sbuf[k, pl.ds(g * 16, 16)]
                dstv = dbuf[k, pl.ds(g * 16, 16)]
                for cc in range(COLS):
                    vals = plsc.load_gather(xslab, [col_ids[cc], srcv])
                    plsc.addupdate_scatter(acc, [col_ids[cc], dstv], vals)

    _fire(0, s0, d0, sem_s0, sem_d0)
    _fire(1, s1, d1, sem_s1, sem_d1)

    def pair(t, _):
        j = t * 2
        _wait(s0, d0, sem_s0, sem_d0)
        _process(s0, d0)

        @pl.when(t < N_SLABS // 2 - 1)
        def _():
            _fire(j + 2, s0, d0, sem_s0, sem_d0)

        _wait(s1, d1, sem_s1, sem_d1)
        _process(s1, d1)

        @pl.when(t < N_SLABS // 2 - 1)
        def _():
            _fire(j + 3, s1, d1, sem_s1, sem_d1)

        return 0

    lax.fori_loop(0, N_SLABS // 2, pair, 0)

    pltpu.sync_copy(acc, outT_hbm.at[q])


_colsum = pl.kernel(
    _colsum_body,
    out_type=jax.ShapeDtypeStruct((NW, COLS, XN), jnp.float32),
    mesh=plsc.VectorSubcoreMesh(core_axis_name="c", subcore_axis_name="s"),
    compiler_params=pltpu.CompilerParams(needs_layout_passes=False),
    scratch_types=[
        pltpu.VMEM((COLS, XN), jnp.float32),
        pltpu.VMEM((COLS, XN), jnp.float32),
        pltpu.VMEM((IDX_SLAB, CH), jnp.int32),
        pltpu.VMEM((IDX_SLAB, CH), jnp.int32),
        pltpu.VMEM((IDX_SLAB, CH), jnp.int32),
        pltpu.VMEM((IDX_SLAB, CH), jnp.int32),
        pltpu.SemaphoreType.DMA,
        pltpu.SemaphoreType.DMA,
        pltpu.SemaphoreType.DMA,
        pltpu.SemaphoreType.DMA,
    ],
)


def _embed_body(xg, xl, xo, xe,
                wg1, bg1, wg2, bg2, wl1, bl1, wl2, bl2,
                wo1, bo1, wo2, bo2, we1, be1, we2, be2, out):
    def mlp2(xt, w1, b1, w2, b2):
        h = _lrelu(jnp.dot(w1[...], xt[...],
                           preferred_element_type=jnp.float32) + b1[...])
        return _lrelu(jnp.dot(w2[...], h,
                              preferred_element_type=jnp.float32) + b2[...])

    out[:, 0:1000] = mlp2(xg, wg1, bg1, wg2, bg2)
    out[:, 1000:2000] = mlp2(xl, wl1, bl1, wl2, bl2)
    out[:, 2000:6000] = mlp2(xo, wo1, bo1, wo2, bo2)
    out[:, 6000:10000] = mlp2(xe, we1, be1, we2, be2)
    out[:, 10000:XN] = jnp.zeros((H, XN - N_NODES), jnp.float32)


_embed = pl.pallas_call(
    _embed_body,
    out_shape=jax.ShapeDtypeStruct((H, XN), jnp.float32),
)


def _layer_body(aggT, xT, wl, bl, wr, out):
    y = (jnp.dot(wl[...], aggT[...], preferred_element_type=jnp.float32)
         + bl[...]
         + jnp.dot(wr[...], xT[...], preferred_element_type=jnp.float32))
    out[...] = _lrelu(y)


_layer = pl.pallas_call(
    _layer_body,
    out_shape=jax.ShapeDtypeStruct((H, XN), jnp.float32),
)


def _final_body(aggT, xT, wl5t, bl, wr5t, out):
    dn = (((0,), (0,)), ((), ()))
    y = (lax.dot_general(aggT[:, 0:N_NODES], wl5t[...], dn,
                         preferred_element_type=jnp.float32)
         + bl[...]
         + lax.dot_general(xT[:, 0:N_NODES], wr5t[...], dn,
                           preferred_element_type=jnp.float32))
    out[...] = jax.nn.sigmoid(y)


_final = pl.pallas_call(
    _final_body,
    out_shape=jax.ShapeDtypeStruct((N_NODES, 1), jnp.float32),
)


def kernel(x_gen, x_load, x_or, x_ex, edge_index, object_ptv,
           W_gen1, b_gen1, W_gen2, b_gen2,
           W_load1, b_load1, W_load2, b_load2,
           W_or1, b_or1, W_or2, b_or2,
           W_ex1, b_ex1, W_ex2, b_ex2,
           Wl_0, bl_0, Wr_0, Wl_1, bl_1, Wr_1, Wl_2, bl_2, Wr_2,
           Wl_3, bl_3, Wr_3, Wl_4, bl_4, Wr_4, Wl_5, bl_5, Wr_5):
    npad = E_PAD - N_EDGES
    src2d = jnp.concatenate(
        [edge_index[0], jnp.zeros((npad,), jnp.int32)]).reshape(CHUNKS_PAD, CH)
    dst2d = jnp.concatenate(
        [edge_index[1], jnp.full((npad,), N_NODES, jnp.int32)]
    ).reshape(CHUNKS_PAD, CH)

    def t(w):
        return jnp.transpose(w)

    def b2(b):
        return b.reshape(-1, 1)

    xT = _embed(t(x_gen), t(x_load), t(x_or), t(x_ex),
                W_gen1, b2(b_gen1), W_gen2, b2(b_gen2),
                W_load1, b2(b_load1), W_load2, b2(b_load2),
                W_or1, b2(b_or1), W_or2, b2(b_or2),
                W_ex1, b2(b_ex1), W_ex2, b2(b_ex2))

    layers = [(Wl_0, bl_0, Wr_0), (Wl_1, bl_1, Wr_1), (Wl_2, bl_2, Wr_2),
              (Wl_3, bl_3, Wr_3), (Wl_4, bl_4, Wr_4)]
    for wl, bl, wr in layers:
        aggT3 = _colsum(xT.reshape(NW, COLS, XN), src2d, dst2d)
        xT = _layer(aggT3.reshape(H, XN), xT, wl, b2(bl), wr)

    aggT3 = _colsum(xT.reshape(NW, COLS, XN), src2d, dst2d)
    return _final(aggT3.reshape(H, XN), xT, t(Wl_5), b2(bl_5), t(Wr_5))
